# Initial kernel scaffold; baseline (speedup 1.0000x reference)
import jax
import jax.numpy as jnp
from jax.experimental import pallas as pl

N_BLK = 4096


def _seg_sum(src, index, n):
    return jax.ops.segment_sum(src, index, num_segments=n)


def _tag_parts(x, ei, n, K):
    src, dst = ei[0], ei[1]
    deg = _seg_sum(jnp.ones(src.shape[0], jnp.float32), dst, n)
    dis = jnp.where(deg > 0, 1.0 / jnp.sqrt(jnp.maximum(deg, 1e-12)), 0.0)
    hs = []
    h = x
    for _ in range(K):
        u = h * dis[:, None] if not hs else h * (dis * dis)[:, None]
        s = _seg_sum(u[src], dst, n)
        h = s * dis[:, None]
        hs.append(h)
    return hs


def _dense_chain_kernel(
    sxh0, sh1, sh2, sh3, aggH, aggI, cnt,
    w2, b2,
    w3rel, b3, w3root, w32rel, b32, w32root,
    w4l, b4, w4r, w42l, b42, w42r,
    wfin, cfin,
    out_ref,
):
    sx = jnp.maximum(
        sxh0[...] @ w2[0] + sh1[...] @ w2[1] + sh2[...] @ w2[2]
        + sh3[...] @ w2[3] + b2[...], 0.0)
    a_h = aggH[...]
    sx = jnp.maximum(a_h @ w3rel[...] + b3[...] + sx @ w3root[...], 0.0)
    sx = jnp.maximum(a_h @ w32rel[...] + b32[...] + sx @ w32root[...], 0.0)
    mean = aggI[...] / jnp.maximum(cnt[...], 1.0)
    sx = jnp.maximum(mean @ w4l[...] + b4[...] + sx @ w4r[...], 0.0)
    sx = jnp.maximum(mean @ w42l[...] + b42[...] + sx @ w42r[...], 0.0)
    out_ref[...] = sx @ wfin[...] + cfin[...]


def _gx_kernel(gxh0, gh1, gh2, w1, b1, out_ref):
    out_ref[...] = jnp.maximum(
        gxh0[...] @ w1[0] + gh1[...] @ w1[1] + gh2[...] @ w1[2] + b1[...], 0.0)


def _row_spec(width):
    return pl.BlockSpec((N_BLK, width), lambda i: (i, 0))


def _full_spec(shape):
    return pl.BlockSpec(shape, lambda i: tuple(0 for _ in shape))


def kernel(game_x, state_x, edge_index_gg, edge_index_ss, edge_index_hist,
           edge_index_in, edge_attr_hist, conv1_w, conv1_b, conv2_w, conv2_b,
           conv3_w_rel, conv3_b_rel, conv3_w_root, conv32_w_rel, conv32_b_rel,
           conv32_w_root, conv4_w_l, conv4_b_l, conv4_w_r, conv42_w_l,
           conv42_b_l, conv42_w_r, lin_w, lin_b, lin_last_w, lin_last_b):
    n_game = game_x.shape[0]
    n_state = state_x.shape[0]

    gh1, gh2 = _tag_parts(game_x, edge_index_gg, n_game, 2)
    sh1, sh2, sh3 = _tag_parts(state_x, edge_index_ss, n_state, 3)

    grid = (pl.cdiv(n_game, N_BLK),)
    gx = pl.pallas_call(
        _gx_kernel,
        grid=grid,
        in_specs=[
            _row_spec(5), _row_spec(5), _row_spec(5),
            _full_spec(conv1_w.shape), _full_spec((32,)),
        ],
        out_specs=_row_spec(32),
        out_shape=jax.ShapeDtypeStruct((n_game, 32), jnp.float32),
    )(game_x, gh1, gh2, conv1_w, conv1_b)

    srcH, dstH = edge_index_hist[0], edge_index_hist[1]
    aggH = _seg_sum(gx[srcH] * edge_attr_hist[:, None], dstH, n_state)
    srcI, dstI = edge_index_in[0], edge_index_in[1]
    aggI = _seg_sum(gx[srcI], dstI, n_state)
    cnt = _seg_sum(jnp.ones(srcI.shape[0], jnp.float32), dstI, n_state)

    wfin = lin_w @ lin_last_w
    cfin = lin_b @ lin_last_w + lin_last_b

    grid = (pl.cdiv(n_state, N_BLK),)
    out = pl.pallas_call(
        _dense_chain_kernel,
        grid=grid,
        in_specs=[
            _row_spec(6), _row_spec(6), _row_spec(6), _row_spec(6),
            _row_spec(32), _row_spec(32), pl.BlockSpec((N_BLK, 1), lambda i: (i, 0)),
            _full_spec(conv2_w.shape), _full_spec((32,)),
            _full_spec((32, 32)), _full_spec((32,)), _full_spec((32, 32)),
            _full_spec((32, 32)), _full_spec((32,)), _full_spec((32, 32)),
            _full_spec((32, 32)), _full_spec((32,)), _full_spec((32, 32)),
            _full_spec((32, 32)), _full_spec((32,)), _full_spec((32, 32)),
            _full_spec((32, 1)), _full_spec((1,)),
        ],
        out_specs=pl.BlockSpec((N_BLK, 1), lambda i: (i, 0)),
        out_shape=jax.ShapeDtypeStruct((n_state, 1), jnp.float32),
    )(state_x, sh1, sh2, sh3, aggH, aggI, cnt[:, None],
      conv2_w, conv2_b,
      conv3_w_rel, conv3_b_rel, conv3_w_root,
      conv32_w_rel, conv32_b_rel, conv32_w_root,
      conv4_w_l, conv4_b_l, conv4_w_r,
      conv42_w_l, conv42_b_l, conv42_w_r,
      wfin, cfin)
    return out


# speedup vs baseline: 2.5415x; 2.5415x over previous
import jax
import jax.numpy as jnp
from jax.experimental import pallas as pl

N_BLK = 4096


def _seg_sum(src, index, n):
    return jax.ops.segment_sum(src, index, num_segments=n)


def _tag_parts(x, ei, n, K):
    src, dst = ei[0], ei[1]
    deg = _seg_sum(jnp.ones(src.shape[0], jnp.float32), dst, n)
    dis = jnp.where(deg > 0, 1.0 / jnp.sqrt(jnp.maximum(deg, 1e-12)), 0.0)
    hs = []
    h = x
    for _ in range(K):
        u = h * dis[:, None]
        s = _seg_sum(u[src], dst, n)
        h = s * dis[:, None]
        hs.append(h)
    return hs


def _dense_chain_kernel(
    sxh0, sh1, sh2, sh3, aggH, aggI, cnt,
    w2, b2,
    w3rel, b3, w3root, w32rel, b32, w32root,
    w4l, b4, w4r, w42l, b42, w42r,
    wfin, cfin,
    out_ref,
):
    sx = jnp.maximum(
        sxh0[...] @ w2[0] + sh1[...] @ w2[1] + sh2[...] @ w2[2]
        + sh3[...] @ w2[3] + b2[...], 0.0)
    a_h = aggH[...]
    sx = jnp.maximum(a_h @ w3rel[...] + b3[...] + sx @ w3root[...], 0.0)
    sx = jnp.maximum(a_h @ w32rel[...] + b32[...] + sx @ w32root[...], 0.0)
    mean = aggI[...] / jnp.maximum(cnt[...], 1.0)
    sx = jnp.maximum(mean @ w4l[...] + b4[...] + sx @ w4r[...], 0.0)
    sx = jnp.maximum(mean @ w42l[...] + b42[...] + sx @ w42r[...], 0.0)
    out_ref[...] = sx @ wfin[...] + cfin[...]


def _gx_kernel(gxh0, gh1, gh2, w1, b1, out_ref):
    out_ref[...] = jnp.maximum(
        gxh0[...] @ w1[0] + gh1[...] @ w1[1] + gh2[...] @ w1[2] + b1[...], 0.0)


def _row_spec(width):
    return pl.BlockSpec((N_BLK, width), lambda i: (i, 0))


def _full_spec(shape):
    return pl.BlockSpec(shape, lambda i: tuple(0 for _ in shape))


def kernel(game_x, state_x, edge_index_gg, edge_index_ss, edge_index_hist,
           edge_index_in, edge_attr_hist, conv1_w, conv1_b, conv2_w, conv2_b,
           conv3_w_rel, conv3_b_rel, conv3_w_root, conv32_w_rel, conv32_b_rel,
           conv32_w_root, conv4_w_l, conv4_b_l, conv4_w_r, conv42_w_l,
           conv42_b_l, conv42_w_r, lin_w, lin_b, lin_last_w, lin_last_b):
    n_game = game_x.shape[0]
    n_state = state_x.shape[0]

    gh1, gh2 = _tag_parts(game_x, edge_index_gg, n_game, 2)
    sh1, sh2, sh3 = _tag_parts(state_x, edge_index_ss, n_state, 3)

    grid = (pl.cdiv(n_game, N_BLK),)
    gx = pl.pallas_call(
        _gx_kernel,
        grid=grid,
        in_specs=[
            _row_spec(5), _row_spec(5), _row_spec(5),
            _full_spec(conv1_w.shape), _full_spec((32,)),
        ],
        out_specs=_row_spec(32),
        out_shape=jax.ShapeDtypeStruct((n_game, 32), jnp.float32),
    )(game_x, gh1, gh2, conv1_w, conv1_b)

    srcH, dstH = edge_index_hist[0], edge_index_hist[1]
    aggH = _seg_sum(gx[srcH] * edge_attr_hist[:, None], dstH, n_state)
    srcI, dstI = edge_index_in[0], edge_index_in[1]
    aggI = _seg_sum(gx[srcI], dstI, n_state)
    cnt = _seg_sum(jnp.ones(srcI.shape[0], jnp.float32), dstI, n_state)

    wfin = lin_w @ lin_last_w
    cfin = lin_b @ lin_last_w + lin_last_b

    grid = (pl.cdiv(n_state, N_BLK),)
    out = pl.pallas_call(
        _dense_chain_kernel,
        grid=grid,
        in_specs=[
            _row_spec(6), _row_spec(6), _row_spec(6), _row_spec(6),
            _row_spec(32), _row_spec(32), pl.BlockSpec((N_BLK, 1), lambda i: (i, 0)),
            _full_spec(conv2_w.shape), _full_spec((32,)),
            _full_spec((32, 32)), _full_spec((32,)), _full_spec((32, 32)),
            _full_spec((32, 32)), _full_spec((32,)), _full_spec((32, 32)),
            _full_spec((32, 32)), _full_spec((32,)), _full_spec((32, 32)),
            _full_spec((32, 32)), _full_spec((32,)), _full_spec((32, 32)),
            _full_spec((32, 1)), _full_spec((1,)),
        ],
        out_specs=pl.BlockSpec((N_BLK, 1), lambda i: (i, 0)),
        out_shape=jax.ShapeDtypeStruct((n_state, 1), jnp.float32),
    )(state_x, sh1, sh2, sh3, aggH, aggI, cnt[:, None],
      conv2_w, conv2_b,
      conv3_w_rel, conv3_b_rel, conv3_w_root,
      conv32_w_rel, conv32_b_rel, conv32_w_root,
      conv4_w_l, conv4_b_l, conv4_w_r,
      conv42_w_l, conv42_b_l, conv42_w_r,
      wfin, cfin)
    return out


# speedup vs baseline: 2.9118x; 1.1457x over previous
import jax
import jax.numpy as jnp
from jax import lax
from jax.experimental import pallas as pl
from jax.experimental.pallas import tpu as pltpu
from jax.experimental.pallas import tpu_sc as plsc

N_BLK = 4096

NT = 16
N_PADN = 100352
SLICE = N_PADN // NT
C_E = 512
EPT_B = 100352
NCH_B = EPT_B // C_E
E_PAD_B = EPT_B * NT


def _chunks(total, step):
    out = []
    off = 0
    while off < total:
        sz = min(step, total - off)
        out.append((off, sz))
        off += sz
    return out


def _pad_edges(src, dst, e_pad, extra=None):
    e = src.shape[0]
    pad = e_pad - e
    fill = jnp.full((pad,), N_PADN - 1, dtype=src.dtype)
    src = jnp.concatenate([src, fill])
    dst = jnp.concatenate([dst, fill])
    if extra is not None:
        extra = jnp.concatenate([extra, jnp.zeros((pad,), extra.dtype)])
        return src, dst, extra
    return src, dst


def _splat(vec_ref, e):
    return plsc.load_gather(vec_ref, [jnp.full((16,), e, jnp.int32)])


def _zero_vmem_rows(ref, nrows):
    def body(i, _):
        ref[i] = jnp.zeros((16,), jnp.float32)
        return _
    lax.fori_loop(0, nrows, body, None)


def _s2_body(gx_flat, srcH, dstH, ewH, srcI, dstI,
             aggH_out, aggI_out, cnt_out,
             acc, cnt_sp, src_v, dst_v, ew_v, rows_v, zrows, zflat, ones_v,
             sem):
    c = lax.axis_index("c")
    s = lax.axis_index("s")
    goff = c * N_PADN

    _zero_vmem_rows(zrows, 128)
    for i in range(32):
        zflat[pl.ds(i * 16, 16)] = jnp.zeros((16,), jnp.float32)
    for i in range(8):
        ones_v[pl.ds(i * 16, 16)] = jnp.ones((16,), jnp.float32)

    def zero_acc():
        for off, sz in _chunks(SLICE, 128):
            pltpu.sync_copy(zrows.at[pl.ds(0, sz)],
                            acc.at[pl.ds(s * SLICE + off, sz)])

    def zero_cnt():
        for off, sz in _chunks(SLICE, 512):
            pltpu.sync_copy(zflat.at[pl.ds(0, sz)],
                            cnt_sp.at[pl.ds(s * SLICE + off, sz)])

    def dump_acc(out_hbm):
        for off, sz in _chunks(SLICE, 512):
            pltpu.sync_copy(acc.at[pl.ds(s * SLICE + off, sz)],
                            out_hbm.at[pl.ds(goff + s * SLICE + off, sz)])

    def edge_pass(src_hbm, dst_hbm, ew_hbm, count):
        def chunk(k, _):
            rbase = s * (EPT_B // 128) + k * 4
            pltpu.sync_copy(src_hbm.at[pl.ds(rbase, 4)], src_v)
            pltpu.sync_copy(dst_hbm.at[pl.ds(rbase, 4)], dst_v)
            if ew_hbm is not None:
                pltpu.sync_copy(ew_hbm.at[pl.ds(rbase * 128, C_E)], ew_v)
            for g in range(32):
                r, co = g // 8, (g % 8) * 16
                src_v[r, pl.ds(co, 16)] = src_v[r, pl.ds(co, 16)] + goff
            descs = [pltpu.async_copy(gx_flat.at[src_v.at[r]],
                                      rows_v.at[pl.ds(r * 128, 128)], sem)
                     for r in range(4)]
            for d in descs:
                d.wait()
            if ew_hbm is not None:
                def scale(g, _):
                    for jj in range(16):
                        e = g * 16 + jj
                        w = _splat(ew_v, e)
                        rows_v[e] = rows_v[e] * w
                    return _
                lax.fori_loop(0, 32, scale, None)
            if count:
                def do_count():
                    for r in range(4):
                        pltpu.sync_copy(ones_v, cnt_sp.at[dst_v.at[r]],
                                        add=True)
                pl.when(c == 0)(do_count)
            for r in range(4):
                pltpu.sync_copy(rows_v.at[pl.ds(r * 128, 128)],
                                acc.at[dst_v.at[r]], add=True)
            return _
        lax.fori_loop(0, NCH_B, chunk, None)

    zero_acc()
    zero_cnt()
    plsc.subcore_barrier()
    edge_pass(srcH, dstH, ewH, count=False)
    plsc.subcore_barrier()
    dump_acc(aggH_out)
    zero_acc()
    plsc.subcore_barrier()
    edge_pass(srcI, dstI, None, count=True)
    plsc.subcore_barrier()
    dump_acc(aggI_out)

    def dump_cnt():
        pltpu.sync_copy(cnt_sp.at[pl.ds(s * SLICE, SLICE)],
                        cnt_out.at[pl.ds(s * SLICE, SLICE)])
    pl.when(c == 0)(dump_cnt)


def _sc_bipartite_aggs(gx_planes, srcH, dstH, ewH, srcI, dstI):
    srcH, dstH, ewH = _pad_edges(srcH, dstH, E_PAD_B, ewH)
    srcI, dstI = _pad_edges(srcI, dstI, E_PAD_B)
    r2 = lambda a: a.reshape(E_PAD_B // 128, 128)
    mesh = plsc.VectorSubcoreMesh(core_axis_name="c", subcore_axis_name="s")
    f32 = jnp.float32
    aggH, aggI, cnt = pl.kernel(
        _s2_body,
        out_type=[
            jax.ShapeDtypeStruct((2 * N_PADN, 16), f32),
            jax.ShapeDtypeStruct((2 * N_PADN, 16), f32),
            jax.ShapeDtypeStruct((N_PADN,), f32),
        ],
        mesh=mesh,
        compiler_params=pltpu.CompilerParams(
            needs_layout_passes=False, use_tc_tiling_on_sc=False),
        scratch_types=[
            pltpu.VMEM_SHARED((N_PADN, 16), f32),
            pltpu.VMEM_SHARED((N_PADN,), f32),
            pltpu.VMEM((4, 128), jnp.int32),
            pltpu.VMEM((4, 128), jnp.int32),
            pltpu.VMEM((C_E,), f32),
            pltpu.VMEM((C_E, 16), f32),
            pltpu.VMEM((128, 16), f32),
            pltpu.VMEM((512,), f32),
            pltpu.VMEM((128,), f32),
            pltpu.SemaphoreType.DMA,
        ],
    )(gx_planes, r2(srcH), r2(dstH), ewH, r2(srcI), r2(dstI))
    return aggH, aggI, cnt


def _seg_sum(src, index, n):
    return jax.ops.segment_sum(src, index, num_segments=n)


def _tag_parts(x, ei, n, K):
    src, dst = ei[0], ei[1]
    deg = _seg_sum(jnp.ones(src.shape[0], jnp.float32), dst, n)
    dis = jnp.where(deg > 0, 1.0 / jnp.sqrt(jnp.maximum(deg, 1e-12)), 0.0)
    hs = []
    h = x
    for _ in range(K):
        u = h * dis[:, None]
        s = _seg_sum(u[src], dst, n)
        h = s * dis[:, None]
        hs.append(h)
    return hs


def _dense_chain_kernel(
    sxh0, sh1, sh2, sh3, aggH0, aggH1, aggI0, aggI1, cnt_in,
    w2, b2,
    w3rel, b3, w3root, w32rel, b32, w32root,
    w4l, b4, w4r, w42l, b42, w42r,
    wfin, cfin,
    out_ref,
):
    sx = jnp.maximum(
        sxh0[...] @ w2[0] + sh1[...] @ w2[1] + sh2[...] @ w2[2]
        + sh3[...] @ w2[3] + b2[...], 0.0)
    a_h = jnp.concatenate([aggH0[...], aggH1[...]], axis=1)
    sx = jnp.maximum(a_h @ w3rel[...] + b3[...] + sx @ w3root[...], 0.0)
    sx = jnp.maximum(a_h @ w32rel[...] + b32[...] + sx @ w32root[...], 0.0)
    a_i = jnp.concatenate([aggI0[...], aggI1[...]], axis=1)
    mean = a_i / jnp.maximum(cnt_in[...], 1.0)
    sx = jnp.maximum(mean @ w4l[...] + b4[...] + sx @ w4r[...], 0.0)
    sx = jnp.maximum(mean @ w42l[...] + b42[...] + sx @ w42r[...], 0.0)
    out_ref[...] = sx @ wfin[...] + cfin[...]


def _gx_kernel(gxh0, gh1, gh2, w1, b1, out_ref):
    c = pl.program_id(0)
    gx = jnp.maximum(
        gxh0[...] @ w1[0] + gh1[...] @ w1[1] + gh2[...] @ w1[2] + b1[...], 0.0)
    out_ref[...] = jnp.where(c == 0, gx[:, :16], gx[:, 16:])


def _row_spec(width):
    return pl.BlockSpec((N_BLK, width), lambda i: (i, 0))


def _full_spec(shape):
    return pl.BlockSpec(shape, lambda i: tuple(0 for _ in shape))


def kernel(game_x, state_x, edge_index_gg, edge_index_ss, edge_index_hist,
           edge_index_in, edge_attr_hist, conv1_w, conv1_b, conv2_w, conv2_b,
           conv3_w_rel, conv3_b_rel, conv3_w_root, conv32_w_rel, conv32_b_rel,
           conv32_w_root, conv4_w_l, conv4_b_l, conv4_w_r, conv42_w_l,
           conv42_b_l, conv42_w_r, lin_w, lin_b, lin_last_w, lin_last_b):
    n_game = game_x.shape[0]
    n_state = state_x.shape[0]

    gh1, gh2 = _tag_parts(game_x, edge_index_gg, n_game, 2)
    sh1, sh2, sh3 = _tag_parts(state_x, edge_index_ss, n_state, 3)

    pad = lambda a: jnp.pad(a, ((0, N_PADN - a.shape[0]), (0, 0)))
    gxp, gh1p, gh2p = pad(game_x), pad(gh1), pad(gh2)

    blk1 = 2048
    nb = N_PADN // blk1
    rs = lambda w: pl.BlockSpec((blk1, w), lambda c, i: (i, 0))
    fs = lambda shape: pl.BlockSpec(shape, lambda c, i: tuple(0 for _ in shape))
    gx_planes = pl.pallas_call(
        _gx_kernel,
        grid=(2, nb),
        in_specs=[rs(5), rs(5), rs(5), fs(conv1_w.shape), fs((32,))],
        out_specs=pl.BlockSpec((blk1, 16), lambda c, i: (c * nb + i, 0)),
        out_shape=jax.ShapeDtypeStruct((2 * N_PADN, 16), jnp.float32),
    )(gxp, gh1p, gh2p, conv1_w, conv1_b)

    aggH, aggI, cnt = _sc_bipartite_aggs(
        gx_planes,
        edge_index_hist[0], edge_index_hist[1], edge_attr_hist,
        edge_index_in[0], edge_index_in[1])
    cnt = cnt[:, None]

    wfin = lin_w @ lin_last_w
    cfin = lin_b @ lin_last_w + lin_last_b

    sxp, sh1p, sh2p, sh3p = pad(state_x), pad(sh1), pad(sh2), pad(sh3)
    rs2 = lambda w: pl.BlockSpec((blk1, w), lambda i: (i, 0))
    fs2 = lambda shape: pl.BlockSpec(shape, lambda i: tuple(0 for _ in shape))
    out = pl.pallas_call(
        _dense_chain_kernel,
        grid=(nb,),
        in_specs=[
            rs2(6), rs2(6), rs2(6), rs2(6),
            rs2(16), rs2(16), rs2(16), rs2(16),
            pl.BlockSpec((blk1, 1), lambda i: (i, 0)),
            fs2(conv2_w.shape), fs2((32,)),
            fs2((32, 32)), fs2((32,)), fs2((32, 32)),
            fs2((32, 32)), fs2((32,)), fs2((32, 32)),
            fs2((32, 32)), fs2((32,)), fs2((32, 32)),
            fs2((32, 32)), fs2((32,)), fs2((32, 32)),
            fs2((32, 1)), fs2((1,)),
        ],
        out_specs=pl.BlockSpec((blk1, 1), lambda i: (i, 0)),
        out_shape=jax.ShapeDtypeStruct((N_PADN, 1), jnp.float32),
    )(sxp, sh1p, sh2p, sh3p,
      aggH[:N_PADN], aggH[N_PADN:], aggI[:N_PADN], aggI[N_PADN:], cnt,
      conv2_w, conv2_b,
      conv3_w_rel, conv3_b_rel, conv3_w_root,
      conv32_w_rel, conv32_b_rel, conv32_w_root,
      conv4_w_l, conv4_b_l, conv4_w_r,
      conv42_w_l, conv42_b_l, conv42_w_r,
      wfin, cfin)
    return out[:n_state]


# speedup vs baseline: 24.5510x; 8.4316x over previous
import jax
import jax.numpy as jnp
from jax import lax
from jax.experimental import pallas as pl
from jax.experimental.pallas import tpu as pltpu
from jax.experimental.pallas import tpu_sc as plsc

N_BLK = 4096

NT = 16
N_PADN = 100352
SLICE = N_PADN // NT
C_E = 512
EPT_B = 100352
NCH_B = EPT_B // C_E
E_PAD_B = EPT_B * NT

EPT_GG = 200192
NCH_GG = EPT_GG // C_E
E_PAD_GG = EPT_GG * NT
EPT_SS = EPT_B
NCH_SS = NCH_B
E_PAD_SS = E_PAD_B


def _chunks(total, step):
    out = []
    off = 0
    while off < total:
        sz = min(step, total - off)
        out.append((off, sz))
        off += sz
    return out


def _pad_edges(src, dst, e_pad, extra=None):
    e = src.shape[0]
    pad = e_pad - e
    fill = jnp.full((pad,), N_PADN - 1, dtype=src.dtype)
    src = jnp.concatenate([src, fill])
    dst = jnp.concatenate([dst, fill])
    if extra is not None:
        extra = jnp.concatenate([extra, jnp.zeros((pad,), extra.dtype)])
        return src, dst, extra
    return src, dst


def _splat(vec_ref, e):
    return plsc.load_gather(vec_ref, [jnp.full((16,), e, jnp.int32)])


def _zero_vmem_rows(ref, nrows):
    def body(i, _):
        ref[i] = jnp.zeros((16,), jnp.float32)
        return _
    lax.fori_loop(0, nrows, body, None)


def _fast_rsqrt(d):
    iv = plsc.bitcast(d, jnp.int32)
    iv = 0x5F3759DF - lax.shift_right_logical(iv, 1)
    y = plsc.bitcast(iv, jnp.float32)
    for _ in range(3):
        y = y * (1.5 - 0.5 * d * y * y)
    return jnp.where(d > 0, y, 0.0)


def _s1_body(x_flat, src_all, dst_all,
             h_out, u_out,
             acc, deg_sp, src_v, dst_v, rows_v, zrows, zflat, ones_v, dis_v,
             sem):
    c = lax.axis_index("c")
    s = lax.axis_index("s")
    goff = c * N_PADN
    ept_rows = jnp.where(c == 0, EPT_GG // 128, EPT_SS // 128)
    ebase_rows = c * (E_PAD_GG // 128) + s * ept_rows
    nch12 = jnp.where(c == 0, NCH_GG, NCH_SS)
    nch3 = jnp.where(c == 0, 0, NCH_SS)

    _zero_vmem_rows(zrows, 128)
    for i in range(32):
        zflat[pl.ds(i * 16, 16)] = jnp.zeros((16,), jnp.float32)
    for i in range(8):
        ones_v[pl.ds(i * 16, 16)] = jnp.ones((16,), jnp.float32)

    def zero_acc():
        for off, sz in _chunks(SLICE, 128):
            pltpu.sync_copy(zrows.at[pl.ds(0, sz)],
                            acc.at[pl.ds(s * SLICE + off, sz)])

    def zero_deg():
        for off, sz in _chunks(SLICE, 512):
            pltpu.sync_copy(zflat.at[pl.ds(0, sz)],
                            deg_sp.at[pl.ds(s * SLICE + off, sz)])

    def load_idx(j, need_src):
        rbase = ebase_rows + j * 4
        if need_src:
            pltpu.sync_copy(src_all.at[pl.ds(rbase, 4)], src_v)
            for g in range(32):
                r, co = g // 8, (g % 8) * 16
                src_v[r, pl.ds(co, 16)] = src_v[r, pl.ds(co, 16)] + goff
        pltpu.sync_copy(dst_all.at[pl.ds(rbase, 4)], dst_v)

    zero_deg()
    plsc.subcore_barrier()

    def deg_chunk(j, carry):
        load_idx(j, need_src=False)
        for r in range(4):
            pltpu.sync_copy(ones_v, deg_sp.at[dst_v.at[r]], add=True)
        return carry
    lax.fori_loop(0, nch12, deg_chunk, None)
    plsc.subcore_barrier()

    pltpu.sync_copy(deg_sp.at[pl.ds(s * SLICE, SLICE)], dis_v)

    def dis_step(i, carry):
        d = dis_v[pl.ds(i * 16, 16)]
        dis_v[pl.ds(i * 16, 16)] = _fast_rsqrt(d)
        return carry
    lax.fori_loop(0, SLICE // 16, dis_step, None)

    def scale_rows(sz):
        def step(j, carry):
            w = _splat(dis_v, carry + j)
            rows_v[j] = rows_v[j] * w
            return carry
        return step

    for off, sz in _chunks(SLICE, 512):
        pltpu.sync_copy(x_flat.at[pl.ds(goff + s * SLICE + off, sz)],
                        rows_v.at[pl.ds(0, sz)])
        lax.fori_loop(0, sz, scale_rows(sz), off)
        pltpu.sync_copy(rows_v.at[pl.ds(0, sz)],
                        u_out.at[pl.ds(goff + s * SLICE + off, sz)])
    plsc.subcore_barrier()

    for k in range(3):
        zero_acc()
        plsc.subcore_barrier()

        def hop_chunk(j, carry):
            load_idx(j, need_src=True)
            descs = [pltpu.async_copy(u_out.at[src_v.at[r]],
                                      rows_v.at[pl.ds(r * 128, 128)], sem)
                     for r in range(4)]
            for d in descs:
                d.wait()
            for r in range(4):
                pltpu.sync_copy(rows_v.at[pl.ds(r * 128, 128)],
                                acc.at[dst_v.at[r]], add=True)
            return carry
        lax.fori_loop(0, nch12 if k < 2 else nch3, hop_chunk, None)
        plsc.subcore_barrier()

        hbase = (3 * c + k) * N_PADN
        for off, sz in _chunks(SLICE, 512):
            pltpu.sync_copy(acc.at[pl.ds(s * SLICE + off, sz)],
                            rows_v.at[pl.ds(0, sz)])
            lax.fori_loop(0, sz, scale_rows(sz), off)
            pltpu.sync_copy(rows_v.at[pl.ds(0, sz)],
                            h_out.at[pl.ds(hbase + s * SLICE + off, sz)])
            if k < 2:
                lax.fori_loop(0, sz, scale_rows(sz), off)
                pltpu.sync_copy(rows_v.at[pl.ds(0, sz)],
                                u_out.at[pl.ds(goff + s * SLICE + off, sz)])
        plsc.subcore_barrier()


def _sc_tag_hops(x_flat, src_all, dst_all):
    mesh = plsc.VectorSubcoreMesh(core_axis_name="c", subcore_axis_name="s")
    f32 = jnp.float32
    etot = E_PAD_GG + E_PAD_SS
    r2 = lambda a: a.reshape(etot // 128, 128)
    h_all, _u = pl.kernel(
        _s1_body,
        out_type=[
            jax.ShapeDtypeStruct((6 * N_PADN, 16), f32),
            jax.ShapeDtypeStruct((2 * N_PADN, 16), f32),
        ],
        mesh=mesh,
        compiler_params=pltpu.CompilerParams(
            needs_layout_passes=False, use_tc_tiling_on_sc=False),
        scratch_types=[
            pltpu.VMEM_SHARED((N_PADN, 16), f32),
            pltpu.VMEM_SHARED((N_PADN,), f32),
            pltpu.VMEM((4, 128), jnp.int32),
            pltpu.VMEM((4, 128), jnp.int32),
            pltpu.VMEM((C_E, 16), f32),
            pltpu.VMEM((128, 16), f32),
            pltpu.VMEM((512,), f32),
            pltpu.VMEM((128,), f32),
            pltpu.VMEM((SLICE,), f32),
            pltpu.SemaphoreType.DMA,
        ],
    )(x_flat, r2(src_all), r2(dst_all))
    return h_all


def _s2_body(gx_flat, srcH, dstH, ewH, srcI, dstI,
             aggH_out, aggI_out, cnt_out,
             acc, cnt_sp, src_v, dst_v, ew_v, rows_v, zrows, zflat, ones_v,
             sem):
    c = lax.axis_index("c")
    s = lax.axis_index("s")
    goff = c * N_PADN

    _zero_vmem_rows(zrows, 128)
    for i in range(32):
        zflat[pl.ds(i * 16, 16)] = jnp.zeros((16,), jnp.float32)
    for i in range(8):
        ones_v[pl.ds(i * 16, 16)] = jnp.ones((16,), jnp.float32)

    def zero_acc():
        for off, sz in _chunks(SLICE, 128):
            pltpu.sync_copy(zrows.at[pl.ds(0, sz)],
                            acc.at[pl.ds(s * SLICE + off, sz)])

    def zero_cnt():
        for off, sz in _chunks(SLICE, 512):
            pltpu.sync_copy(zflat.at[pl.ds(0, sz)],
                            cnt_sp.at[pl.ds(s * SLICE + off, sz)])

    def dump_acc(out_hbm):
        for off, sz in _chunks(SLICE, 512):
            pltpu.sync_copy(acc.at[pl.ds(s * SLICE + off, sz)],
                            out_hbm.at[pl.ds(goff + s * SLICE + off, sz)])

    def edge_pass(src_hbm, dst_hbm, ew_hbm, count):
        def chunk(k, _):
            rbase = s * (EPT_B // 128) + k * 4
            pltpu.sync_copy(src_hbm.at[pl.ds(rbase, 4)], src_v)
            pltpu.sync_copy(dst_hbm.at[pl.ds(rbase, 4)], dst_v)
            if ew_hbm is not None:
                pltpu.sync_copy(ew_hbm.at[pl.ds(rbase * 128, C_E)], ew_v)
            for g in range(32):
                r, co = g // 8, (g % 8) * 16
                src_v[r, pl.ds(co, 16)] = src_v[r, pl.ds(co, 16)] + goff
            descs = [pltpu.async_copy(gx_flat.at[src_v.at[r]],
                                      rows_v.at[pl.ds(r * 128, 128)], sem)
                     for r in range(4)]
            for d in descs:
                d.wait()
            if ew_hbm is not None:
                def scale(g, _):
                    for jj in range(16):
                        e = g * 16 + jj
                        w = _splat(ew_v, e)
                        rows_v[e] = rows_v[e] * w
                    return _
                lax.fori_loop(0, 32, scale, None)
            if count:
                def do_count():
                    for r in range(4):
                        pltpu.sync_copy(ones_v, cnt_sp.at[dst_v.at[r]],
                                        add=True)
                pl.when(c == 0)(do_count)
            for r in range(4):
                pltpu.sync_copy(rows_v.at[pl.ds(r * 128, 128)],
                                acc.at[dst_v.at[r]], add=True)
            return _
        lax.fori_loop(0, NCH_B, chunk, None)

    zero_acc()
    zero_cnt()
    plsc.subcore_barrier()
    edge_pass(srcH, dstH, ewH, count=False)
    plsc.subcore_barrier()
    dump_acc(aggH_out)
    zero_acc()
    plsc.subcore_barrier()
    edge_pass(srcI, dstI, None, count=True)
    plsc.subcore_barrier()
    dump_acc(aggI_out)

    def dump_cnt():
        pltpu.sync_copy(cnt_sp.at[pl.ds(s * SLICE, SLICE)],
                        cnt_out.at[pl.ds(s * SLICE, SLICE)])
    pl.when(c == 0)(dump_cnt)


def _sc_bipartite_aggs(gx_planes, srcH, dstH, ewH, srcI, dstI):
    srcH, dstH, ewH = _pad_edges(srcH, dstH, E_PAD_B, ewH)
    srcI, dstI = _pad_edges(srcI, dstI, E_PAD_B)
    r2 = lambda a: a.reshape(E_PAD_B // 128, 128)
    mesh = plsc.VectorSubcoreMesh(core_axis_name="c", subcore_axis_name="s")
    f32 = jnp.float32
    aggH, aggI, cnt = pl.kernel(
        _s2_body,
        out_type=[
            jax.ShapeDtypeStruct((2 * N_PADN, 16), f32),
            jax.ShapeDtypeStruct((2 * N_PADN, 16), f32),
            jax.ShapeDtypeStruct((N_PADN,), f32),
        ],
        mesh=mesh,
        compiler_params=pltpu.CompilerParams(
            needs_layout_passes=False, use_tc_tiling_on_sc=False),
        scratch_types=[
            pltpu.VMEM_SHARED((N_PADN, 16), f32),
            pltpu.VMEM_SHARED((N_PADN,), f32),
            pltpu.VMEM((4, 128), jnp.int32),
            pltpu.VMEM((4, 128), jnp.int32),
            pltpu.VMEM((C_E,), f32),
            pltpu.VMEM((C_E, 16), f32),
            pltpu.VMEM((128, 16), f32),
            pltpu.VMEM((512,), f32),
            pltpu.VMEM((128,), f32),
            pltpu.SemaphoreType.DMA,
        ],
    )(gx_planes, r2(srcH), r2(dstH), ewH, r2(srcI), r2(dstI))
    return aggH, aggI, cnt


def _seg_sum(src, index, n):
    return jax.ops.segment_sum(src, index, num_segments=n)


def _tag_parts(x, ei, n, K):
    src, dst = ei[0], ei[1]
    deg = _seg_sum(jnp.ones(src.shape[0], jnp.float32), dst, n)
    dis = jnp.where(deg > 0, 1.0 / jnp.sqrt(jnp.maximum(deg, 1e-12)), 0.0)
    hs = []
    h = x
    for _ in range(K):
        u = h * dis[:, None]
        s = _seg_sum(u[src], dst, n)
        h = s * dis[:, None]
        hs.append(h)
    return hs


def _dense_chain_kernel(
    sxh0, sh1, sh2, sh3, aggH0, aggH1, aggI0, aggI1, cnt_in,
    w2, b2,
    w3rel, b3, w3root, w32rel, b32, w32root,
    w4l, b4, w4r, w42l, b42, w42r,
    wfin, cfin,
    out_ref,
):
    sx = jnp.maximum(
        sxh0[...] @ w2[0] + sh1[...] @ w2[1] + sh2[...] @ w2[2]
        + sh3[...] @ w2[3] + b2[...], 0.0)
    a_h = jnp.concatenate([aggH0[...], aggH1[...]], axis=1)
    sx = jnp.maximum(a_h @ w3rel[...] + b3[...] + sx @ w3root[...], 0.0)
    sx = jnp.maximum(a_h @ w32rel[...] + b32[...] + sx @ w32root[...], 0.0)
    a_i = jnp.concatenate([aggI0[...], aggI1[...]], axis=1)
    mean = a_i / jnp.maximum(cnt_in[...], 1.0)
    sx = jnp.maximum(mean @ w4l[...] + b4[...] + sx @ w4r[...], 0.0)
    sx = jnp.maximum(mean @ w42l[...] + b42[...] + sx @ w42r[...], 0.0)
    out_ref[...] = sx @ wfin[...] + cfin[...]


def _gx_kernel(gxh0, gh1, gh2, w1, b1, out_ref):
    c = pl.program_id(0)
    gx = jnp.maximum(
        gxh0[...] @ w1[0] + gh1[...] @ w1[1] + gh2[...] @ w1[2] + b1[...], 0.0)
    out_ref[...] = jnp.where(c == 0, gx[:, :16], gx[:, 16:])


def _row_spec(width):
    return pl.BlockSpec((N_BLK, width), lambda i: (i, 0))


def _full_spec(shape):
    return pl.BlockSpec(shape, lambda i: tuple(0 for _ in shape))


def kernel(game_x, state_x, edge_index_gg, edge_index_ss, edge_index_hist,
           edge_index_in, edge_attr_hist, conv1_w, conv1_b, conv2_w, conv2_b,
           conv3_w_rel, conv3_b_rel, conv3_w_root, conv32_w_rel, conv32_b_rel,
           conv32_w_root, conv4_w_l, conv4_b_l, conv4_w_r, conv42_w_l,
           conv42_b_l, conv42_w_r, lin_w, lin_b, lin_last_w, lin_last_b):
    n_game = game_x.shape[0]
    n_state = state_x.shape[0]

    x_flat = jnp.zeros((2 * N_PADN, 16), jnp.float32)
    x_flat = x_flat.at[:n_game, :5].set(game_x)
    x_flat = x_flat.at[N_PADN:N_PADN + n_state, :6].set(state_x)
    srcg, dstg = _pad_edges(edge_index_gg[0], edge_index_gg[1], E_PAD_GG)
    srcs, dsts = _pad_edges(edge_index_ss[0], edge_index_ss[1], E_PAD_SS)
    h_all = _sc_tag_hops(x_flat,
                         jnp.concatenate([srcg, srcs]),
                         jnp.concatenate([dstg, dsts]))
    gh1p, gh2p = h_all[:N_PADN], h_all[N_PADN:2 * N_PADN]
    sh1p = h_all[3 * N_PADN:4 * N_PADN]
    sh2p = h_all[4 * N_PADN:5 * N_PADN]
    sh3p = h_all[5 * N_PADN:]
    w1p = jnp.zeros((3, 16, 32), jnp.float32).at[:, :5].set(conv1_w)
    w2p = jnp.zeros((4, 16, 32), jnp.float32).at[:, :6].set(conv2_w)

    blk1 = 2048
    nb = N_PADN // blk1
    rs = lambda w: pl.BlockSpec((blk1, w), lambda c, i: (i, 0))
    fs = lambda shape: pl.BlockSpec(shape, lambda c, i: tuple(0 for _ in shape))
    gx_planes = pl.pallas_call(
        _gx_kernel,
        grid=(2, nb),
        in_specs=[rs(16), rs(16), rs(16), fs(w1p.shape), fs((32,))],
        out_specs=pl.BlockSpec((blk1, 16), lambda c, i: (c * nb + i, 0)),
        out_shape=jax.ShapeDtypeStruct((2 * N_PADN, 16), jnp.float32),
    )(x_flat[:N_PADN], gh1p, gh2p, w1p, conv1_b)

    aggH, aggI, cnt = _sc_bipartite_aggs(
        gx_planes,
        edge_index_hist[0], edge_index_hist[1], edge_attr_hist,
        edge_index_in[0], edge_index_in[1])
    cnt = cnt[:, None]

    wfin = lin_w @ lin_last_w
    cfin = lin_b @ lin_last_w + lin_last_b

    rs2 = lambda w: pl.BlockSpec((blk1, w), lambda i: (i, 0))
    fs2 = lambda shape: pl.BlockSpec(shape, lambda i: tuple(0 for _ in shape))
    out = pl.pallas_call(
        _dense_chain_kernel,
        grid=(nb,),
        in_specs=[
            rs2(16), rs2(16), rs2(16), rs2(16),
            rs2(16), rs2(16), rs2(16), rs2(16),
            pl.BlockSpec((blk1, 1), lambda i: (i, 0)),
            fs2(w2p.shape), fs2((32,)),
            fs2((32, 32)), fs2((32,)), fs2((32, 32)),
            fs2((32, 32)), fs2((32,)), fs2((32, 32)),
            fs2((32, 32)), fs2((32,)), fs2((32, 32)),
            fs2((32, 32)), fs2((32,)), fs2((32, 32)),
            fs2((32, 1)), fs2((1,)),
        ],
        out_specs=pl.BlockSpec((blk1, 1), lambda i: (i, 0)),
        out_shape=jax.ShapeDtypeStruct((N_PADN, 1), jnp.float32),
    )(x_flat[N_PADN:], sh1p, sh2p, sh3p,
      aggH[:N_PADN], aggH[N_PADN:], aggI[:N_PADN], aggI[N_PADN:], cnt,
      w2p, conv2_b,
      conv3_w_rel, conv3_b_rel, conv3_w_root,
      conv32_w_rel, conv32_b_rel, conv32_w_root,
      conv4_w_l, conv4_b_l, conv4_w_r,
      conv42_w_l, conv42_b_l, conv42_w_r,
      wfin, cfin)
    return out[:n_state]
